# baseline (speedup 1.0000x reference)
import jax
import jax.numpy as jnp
from jax import lax
from jax.experimental import pallas as pl
from jax.experimental.pallas import tpu as pltpu
from jax.experimental.pallas import tpu_sc as plsc

ALPHA_BLEND = 0.3

C = 80
NBUF = 4
SIB = 4
DIB = 8


def _sc_accumulate(src_hbm, sidx_hbm, didx_hbm, edge_hbm,
                   sum_out, cnt_out,
                   acc_sh, cnt_sh,
                   sidx_r, didx_r, erows, ones_v, zcnt_v,
                   sidx_sem, didx_sem, lin_sem, gadd_sem, scat_sem):
    N, D = src_hbm.shape
    E = sidx_hbm.shape[0]
    epw = E // 32
    chunks = epw // C
    npad = cnt_sh.shape[0]
    rows_per_tile = npad // 16

    cid = lax.axis_index("c")
    sid = lax.axis_index("s")
    wid = sid * 2 + cid
    ebase = wid * epw

    zero16 = jnp.zeros((16,), jnp.float32)
    one16 = jnp.ones((16,), jnp.float32)

    def zrow(i, carry):
        for j in range(D // 16):
            erows[0, i, pl.ds(j * 16, 16)] = zero16
        return carry
    lax.fori_loop(0, C, zrow, 0)

    def zcnt(i, carry):
        zcnt_v[pl.ds(i * 16, 16)] = zero16
        return carry
    lax.fori_loop(0, rows_per_tile // 16, zcnt, 0)

    for j in range(C // 16):
        ones_v[pl.ds(j * 16, 16)] = one16

    def zacc(k, carry):
        pltpu.async_copy(erows.at[0],
                         acc_sh.at[pl.ds(sid * rows_per_tile + k * C, C)],
                         scat_sem.at[0])
        return carry
    lax.fori_loop(0, rows_per_tile // C, zacc, 0)
    pltpu.async_copy(zcnt_v, cnt_sh.at[pl.ds(sid * rows_per_tile, rows_per_tile)],
                     scat_sem.at[1])

    def zacc_drain(k, carry):
        pltpu.make_async_copy(
            erows.at[0], acc_sh.at[pl.ds(sid * rows_per_tile + k * C, C)],
            scat_sem.at[0]).wait()
        return carry
    lax.fori_loop(0, rows_per_tile // C, zacc_drain, 0)
    pltpu.make_async_copy(
        zcnt_v, cnt_sh.at[pl.ds(sid * rows_per_tile, rows_per_tile)],
        scat_sem.at[1]).wait()

    plsc.subcore_barrier()

    def issue_sidx(j):
        pltpu.async_copy(sidx_hbm.at[pl.ds(ebase + j * C, C)],
                         sidx_r.at[j % SIB], sidx_sem.at[j % SIB])

    def wait_sidx(j):
        pltpu.make_async_copy(sidx_hbm.at[pl.ds(ebase + j * C, C)],
                              sidx_r.at[j % SIB], sidx_sem.at[j % SIB]).wait()

    def issue_didx(j):
        pltpu.async_copy(didx_hbm.at[pl.ds(ebase + j * C, C)],
                         didx_r.at[j % DIB], didx_sem.at[j % DIB])

    def wait_didx(j):
        pltpu.make_async_copy(didx_hbm.at[pl.ds(ebase + j * C, C)],
                              didx_r.at[j % DIB], didx_sem.at[j % DIB]).wait()

    def issue_lin(j, b):
        pltpu.async_copy(edge_hbm.at[pl.ds(ebase + j * C, C)], erows.at[b],
                         lin_sem.at[b])

    def wait_lin(j, b):
        pltpu.make_async_copy(edge_hbm.at[pl.ds(ebase + j * C, C)],
                              erows.at[b], lin_sem.at[b]).wait()

    def issue_gadd(j, b, s):
        pltpu.async_copy(src_hbm.at[sidx_r.at[s]], erows.at[b],
                         gadd_sem.at[b], add=True)

    def wait_gadd(j, b, s):
        pltpu.make_async_copy(src_hbm.at[sidx_r.at[s]], erows.at[b],
                              gadd_sem.at[b]).wait()

    def issue_scat(j, b, s):
        pltpu.async_copy(erows.at[b], acc_sh.at[didx_r.at[s]],
                         scat_sem.at[b], add=True)
        pltpu.async_copy(ones_v, cnt_sh.at[didx_r.at[s]],
                         scat_sem.at[b], add=True)

    def wait_scat(j, b, s):
        pltpu.make_async_copy(erows.at[b], acc_sh.at[didx_r.at[s]],
                              scat_sem.at[b]).wait()
        pltpu.make_async_copy(ones_v, cnt_sh.at[didx_r.at[s]],
                              scat_sem.at[b]).wait()

    for p in range(3):
        issue_sidx(p)
        issue_didx(p)
    issue_lin(0, 0)
    issue_lin(1, 1)
    wait_sidx(0)
    wait_lin(0, 0)
    issue_gadd(0, 0, 0)

    def group(g, carry):
        for b4 in range(NBUF):
            j = g * NBUF + b4
            b, b1, b2 = b4, (b4 + 1) % NBUF, (b4 + 2) % NBUF
            s1 = (b4 + 1) % SIB

            @pl.when(j >= 2)
            def _():
                wait_scat(j - 2, b2, (j - 2) % DIB)

            @pl.when(j + 3 < chunks)
            def _():
                issue_sidx(j + 3)
                issue_didx(j + 3)

            @pl.when(j + 2 < chunks)
            def _():
                issue_lin(j + 2, b2)

            @pl.when(j + 1 < chunks)
            def _():
                wait_lin(j + 1, b1)
                wait_sidx(j + 1)
                issue_gadd(j + 1, b1, s1)

            wait_gadd(j, b, b4 % SIB)
            wait_didx(j)
            issue_scat(j, b, j % DIB)
        return carry
    lax.fori_loop(0, chunks // NBUF, group, 0)

    for j in range((chunks // NBUF) * NBUF, chunks):
        b = j % NBUF
        wait_scat(j - 2, (j - 2) % NBUF, (j - 2) % DIB)
        if j + 1 < chunks:
            wait_lin(j + 1, (j + 1) % NBUF)
            wait_sidx(j + 1)
            issue_gadd(j + 1, (j + 1) % NBUF, (j + 1) % SIB)
        wait_gadd(j, b, j % SIB)
        wait_didx(j)
        issue_scat(j, b, j % DIB)
    for j in range(chunks - 2, chunks):
        wait_scat(j, j % NBUF, j % DIB)

    plsc.subcore_barrier()

    pltpu.async_copy(cnt_sh.at[pl.ds(sid * rows_per_tile, rows_per_tile)],
                     cnt_out.at[cid, pl.ds(sid * rows_per_tile, rows_per_tile)],
                     scat_sem.at[1])

    last_base = 15 * rows_per_tile
    last_rows = N - last_base

    @pl.when(sid < 15)
    def _():
        pltpu.async_copy(acc_sh.at[pl.ds(sid * rows_per_tile, rows_per_tile)],
                         sum_out.at[cid, pl.ds(sid * rows_per_tile, rows_per_tile)],
                         scat_sem.at[0])
        pltpu.make_async_copy(
            acc_sh.at[pl.ds(sid * rows_per_tile, rows_per_tile)],
            sum_out.at[cid, pl.ds(sid * rows_per_tile, rows_per_tile)],
            scat_sem.at[0]).wait()

    @pl.when(sid == 15)
    def _():
        pltpu.async_copy(acc_sh.at[pl.ds(last_base, last_rows)],
                         sum_out.at[cid, pl.ds(last_base, last_rows)],
                         scat_sem.at[0])
        pltpu.make_async_copy(
            acc_sh.at[pl.ds(last_base, last_rows)],
            sum_out.at[cid, pl.ds(last_base, last_rows)],
            scat_sem.at[0]).wait()

    pltpu.make_async_copy(
        cnt_sh.at[pl.ds(sid * rows_per_tile, rows_per_tile)],
        cnt_out.at[cid, pl.ds(sid * rows_per_tile, rows_per_tile)],
        scat_sem.at[1]).wait()


def _tc_combine(sum_ref, cnt_ref, dst_ref, out_ref):
    N = dst_ref.shape[0]
    s = sum_ref[0, :N, :] + sum_ref[1, :N, :]
    c = cnt_ref[0, :N, :] + cnt_ref[1, :N, :]
    mean = s / jnp.maximum(c, 1.0)
    agg = ALPHA_BLEND * dst_ref[...] + (1.0 - ALPHA_BLEND) * mean
    out_ref[...] = jnp.where(c > 0.0, agg, 0.0)


def kernel(src_embedding, dst_embedding, edge_embedding, edge_index):
    N, D = src_embedding.shape
    E = edge_embedding.shape[0]
    npad = ((N + 639) // 640) * 640

    src_idx = edge_index[0].astype(jnp.int32)
    dst_idx = edge_index[1].astype(jnp.int32)

    mesh = plsc.VectorSubcoreMesh(core_axis_name="c", subcore_axis_name="s")
    sc_call = pl.kernel(
        _sc_accumulate,
        out_type=(
            jax.ShapeDtypeStruct((2, N, D), jnp.float32),
            jax.ShapeDtypeStruct((2, npad), jnp.float32),
        ),
        mesh=mesh,
        scratch_types=[
            pltpu.VMEM_SHARED((npad, D), jnp.float32),
            pltpu.VMEM_SHARED((npad,), jnp.float32),
            pltpu.VMEM((SIB, C), jnp.int32),
            pltpu.VMEM((DIB, C), jnp.int32),
            pltpu.VMEM((NBUF, C, D), jnp.float32),
            pltpu.VMEM((C,), jnp.float32),
            pltpu.VMEM((npad // 16,), jnp.float32),
            pltpu.SemaphoreType.DMA((SIB,)),
            pltpu.SemaphoreType.DMA((DIB,)),
            pltpu.SemaphoreType.DMA((NBUF,)),
            pltpu.SemaphoreType.DMA((NBUF,)),
            pltpu.SemaphoreType.DMA((NBUF,)),
        ],
    )
    sums, cnts = sc_call(src_embedding, src_idx, dst_idx, edge_embedding)

    cnts3 = cnts.reshape(2, npad, 1)
    out = pl.pallas_call(
        _tc_combine,
        out_shape=jax.ShapeDtypeStruct((N, D), jnp.float32),
    )(sums, cnts3, dst_embedding)
    return out


# speedup vs baseline: 1.0100x; 1.0100x over previous
import jax
import jax.numpy as jnp
from jax import lax
from jax.experimental import pallas as pl
from jax.experimental.pallas import tpu as pltpu
from jax.experimental.pallas import tpu_sc as plsc

ALPHA_BLEND = 0.3

C = 80
NBUF = 4
SIB = 4
DIB = 8


def _sc_accumulate(src_hbm, sidx_hbm, didx_hbm, edge_hbm,
                   sum_out, cnt_out,
                   acc_sh, cnt_sh,
                   sidx_r, didx_r, erows, ones_v, zcnt_v,
                   sidx_sem, didx_sem, lin_sem, gadd_sem, scat_sem):
    N, D = src_hbm.shape
    E = sidx_hbm.shape[0]
    epw = E // 32
    chunks = epw // C
    npad = cnt_sh.shape[0]
    rows_per_tile = npad // 16

    cid = lax.axis_index("c")
    sid = lax.axis_index("s")
    wid = sid * 2 + cid
    ebase = wid * epw

    zero16 = jnp.zeros((16,), jnp.float32)
    one16 = jnp.ones((16,), jnp.float32)

    def zrow(i, carry):
        for j in range(D // 16):
            erows[0, i, pl.ds(j * 16, 16)] = zero16
        return carry
    lax.fori_loop(0, C, zrow, 0)

    def zcnt(i, carry):
        zcnt_v[pl.ds(i * 16, 16)] = zero16
        return carry
    lax.fori_loop(0, rows_per_tile // 16, zcnt, 0)

    for j in range(C // 16):
        ones_v[pl.ds(j * 16, 16)] = one16

    def zacc(k, carry):
        pltpu.async_copy(erows.at[0],
                         acc_sh.at[pl.ds(sid * rows_per_tile + k * C, C)],
                         scat_sem.at[0])
        return carry
    lax.fori_loop(0, rows_per_tile // C, zacc, 0)
    pltpu.async_copy(zcnt_v, cnt_sh.at[pl.ds(sid * rows_per_tile, rows_per_tile)],
                     scat_sem.at[1])

    def zacc_drain(k, carry):
        pltpu.make_async_copy(
            erows.at[0], acc_sh.at[pl.ds(sid * rows_per_tile + k * C, C)],
            scat_sem.at[0]).wait()
        return carry
    lax.fori_loop(0, rows_per_tile // C, zacc_drain, 0)
    pltpu.make_async_copy(
        zcnt_v, cnt_sh.at[pl.ds(sid * rows_per_tile, rows_per_tile)],
        scat_sem.at[1]).wait()

    plsc.subcore_barrier()

    def issue_sidx(j):
        pltpu.async_copy(sidx_hbm.at[pl.ds(ebase + j * C, C)],
                         sidx_r.at[j % SIB], sidx_sem.at[j % SIB])

    def wait_sidx(j):
        pltpu.make_async_copy(sidx_hbm.at[pl.ds(ebase + j * C, C)],
                              sidx_r.at[j % SIB], sidx_sem.at[j % SIB]).wait()

    def issue_didx(j):
        pltpu.async_copy(didx_hbm.at[pl.ds(ebase + j * C, C)],
                         didx_r.at[j % DIB], didx_sem.at[j % DIB])

    def wait_didx(j):
        pltpu.make_async_copy(didx_hbm.at[pl.ds(ebase + j * C, C)],
                              didx_r.at[j % DIB], didx_sem.at[j % DIB]).wait()

    def issue_lin(j, b):
        pltpu.async_copy(edge_hbm.at[pl.ds(ebase + j * C, C)], erows.at[b],
                         lin_sem.at[b])

    def wait_lin(j, b):
        pltpu.make_async_copy(edge_hbm.at[pl.ds(ebase + j * C, C)],
                              erows.at[b], lin_sem.at[b]).wait()

    def issue_gadd(j, b, s):
        pltpu.async_copy(src_hbm.at[sidx_r.at[s]], erows.at[b],
                         gadd_sem.at[b], add=True)

    def wait_gadd(j, b, s):
        pltpu.make_async_copy(src_hbm.at[sidx_r.at[s]], erows.at[b],
                              gadd_sem.at[b]).wait()

    def issue_scat(j, b, s):
        pltpu.async_copy(erows.at[b], acc_sh.at[didx_r.at[s]],
                         scat_sem.at[b], add=True)
        pltpu.async_copy(ones_v, cnt_sh.at[didx_r.at[s]],
                         scat_sem.at[b], add=True)

    def wait_scat(j, b, s):
        pltpu.make_async_copy(erows.at[b], acc_sh.at[didx_r.at[s]],
                              scat_sem.at[b]).wait()
        pltpu.make_async_copy(ones_v, cnt_sh.at[didx_r.at[s]],
                              scat_sem.at[b]).wait()

    for p in range(3):
        issue_sidx(p)
        issue_didx(p)
    issue_lin(0, 0)
    issue_lin(1, 1)
    wait_sidx(0)
    wait_lin(0, 0)
    issue_gadd(0, 0, 0)

    def group(g, carry):
        for b4 in range(NBUF):
            j = g * NBUF + b4
            b, b1, b2 = b4, (b4 + 1) % NBUF, (b4 + 2) % NBUF
            s1 = (b4 + 1) % SIB

            @pl.when(j >= 2)
            def _():
                wait_scat(j - 2, b2, (j - 2) % DIB)

            @pl.when(j + 3 < chunks)
            def _():
                issue_sidx(j + 3)
                issue_didx(j + 3)

            @pl.when(j + 2 < chunks)
            def _():
                issue_lin(j + 2, b2)

            @pl.when(j + 1 < chunks)
            def _():
                wait_lin(j + 1, b1)
                wait_sidx(j + 1)
                issue_gadd(j + 1, b1, s1)

            wait_gadd(j, b, b4 % SIB)
            wait_didx(j)
            issue_scat(j, b, j % DIB)
        return carry
    lax.fori_loop(0, chunks // NBUF, group, 0)

    for j in range((chunks // NBUF) * NBUF, chunks):
        b = j % NBUF
        wait_scat(j - 2, (j - 2) % NBUF, (j - 2) % DIB)
        if j + 1 < chunks:
            wait_lin(j + 1, (j + 1) % NBUF)
            wait_sidx(j + 1)
            issue_gadd(j + 1, (j + 1) % NBUF, (j + 1) % SIB)
        wait_gadd(j, b, j % SIB)
        wait_didx(j)
        issue_scat(j, b, j % DIB)
    for j in range(chunks - 2, chunks):
        wait_scat(j, j % NBUF, j % DIB)

    plsc.subcore_barrier()

    pltpu.async_copy(cnt_sh.at[pl.ds(sid * rows_per_tile, rows_per_tile)],
                     cnt_out.at[cid, pl.ds(sid * rows_per_tile, rows_per_tile)],
                     scat_sem.at[1])

    last_base = 15 * rows_per_tile
    last_rows = N - last_base

    @pl.when(sid < 15)
    def _():
        pltpu.async_copy(acc_sh.at[pl.ds(sid * rows_per_tile, rows_per_tile)],
                         sum_out.at[cid, pl.ds(sid * rows_per_tile, rows_per_tile)],
                         scat_sem.at[0])
        pltpu.make_async_copy(
            acc_sh.at[pl.ds(sid * rows_per_tile, rows_per_tile)],
            sum_out.at[cid, pl.ds(sid * rows_per_tile, rows_per_tile)],
            scat_sem.at[0]).wait()

    @pl.when(sid == 15)
    def _():
        pltpu.async_copy(acc_sh.at[pl.ds(last_base, last_rows)],
                         sum_out.at[cid, pl.ds(last_base, last_rows)],
                         scat_sem.at[0])
        pltpu.make_async_copy(
            acc_sh.at[pl.ds(last_base, last_rows)],
            sum_out.at[cid, pl.ds(last_base, last_rows)],
            scat_sem.at[0]).wait()

    pltpu.make_async_copy(
        cnt_sh.at[pl.ds(sid * rows_per_tile, rows_per_tile)],
        cnt_out.at[cid, pl.ds(sid * rows_per_tile, rows_per_tile)],
        scat_sem.at[1]).wait()


def _tc_combine(sum_ref, cnt_ref, dst_ref, out_ref):
    s = sum_ref[0] + sum_ref[1]
    c = cnt_ref[0] + cnt_ref[1]
    mean = s / jnp.maximum(c, 1.0)
    agg = ALPHA_BLEND * dst_ref[...] + (1.0 - ALPHA_BLEND) * mean
    out_ref[...] = jnp.where(c > 0.0, agg, 0.0)


def kernel(src_embedding, dst_embedding, edge_embedding, edge_index):
    N, D = src_embedding.shape
    E = edge_embedding.shape[0]
    npad = ((N + 639) // 640) * 640

    src_idx = edge_index[0].astype(jnp.int32)
    dst_idx = edge_index[1].astype(jnp.int32)

    mesh = plsc.VectorSubcoreMesh(core_axis_name="c", subcore_axis_name="s")
    sc_call = pl.kernel(
        _sc_accumulate,
        out_type=(
            jax.ShapeDtypeStruct((2, N, D), jnp.float32),
            jax.ShapeDtypeStruct((2, npad), jnp.float32),
        ),
        mesh=mesh,
        scratch_types=[
            pltpu.VMEM_SHARED((npad, D), jnp.float32),
            pltpu.VMEM_SHARED((npad,), jnp.float32),
            pltpu.VMEM((SIB, C), jnp.int32),
            pltpu.VMEM((DIB, C), jnp.int32),
            pltpu.VMEM((NBUF, C, D), jnp.float32),
            pltpu.VMEM((C,), jnp.float32),
            pltpu.VMEM((npad // 16,), jnp.float32),
            pltpu.SemaphoreType.DMA((SIB,)),
            pltpu.SemaphoreType.DMA((DIB,)),
            pltpu.SemaphoreType.DMA((NBUF,)),
            pltpu.SemaphoreType.DMA((NBUF,)),
            pltpu.SemaphoreType.DMA((NBUF,)),
        ],
    )
    sums, cnts = sc_call(src_embedding, src_idx, dst_idx, edge_embedding)

    cnts3 = cnts[:, :N].reshape(2, N, 1)
    blocks = 5
    R = N // blocks
    out = pl.pallas_call(
        _tc_combine,
        grid=(blocks,),
        in_specs=[
            pl.BlockSpec((2, R, D), lambda i: (0, i, 0)),
            pl.BlockSpec((2, R, 1), lambda i: (0, i, 0)),
            pl.BlockSpec((R, D), lambda i: (i, 0)),
        ],
        out_specs=pl.BlockSpec((R, D), lambda i: (i, 0)),
        out_shape=jax.ShapeDtypeStruct((N, D), jnp.float32),
    )(sums, cnts3, dst_embedding)
    return out
